# baseline (speedup 1.0000x reference)
import functools
import jax
import jax.numpy as jnp
from jax import lax
from jax.experimental import pallas as pl
from jax.experimental.pallas import tpu as pltpu
from jax.experimental.pallas import tpu_sc as plsc

NUM_USERS = 5000
NUM_ITEMS = 5000
N = NUM_USERS + NUM_ITEMS
D = 128
NUM_LAYERS = 3
E = 320000

NC, NS = 2, 16
NW = NC * NS
CH = 128
NCHT = 2560
EPAD = NCHT * CH
KT = NCHT // NW
KPART = 40
NP = 10240
RPT = NP // NS

_mesh = plsc.VectorSubcoreMesh(
    core_axis_name="c", subcore_axis_name="s", num_cores=NC, num_subcores=NS
)


def _zero16():
    return jnp.zeros((16,), jnp.float32)


@functools.partial(
    pl.kernel,
    out_type=jax.ShapeDtypeStruct((NC, NS, RPT), jnp.float32),
    mesh=_mesh,
    scratch_types=[
        pltpu.VMEM_SHARED((NP,), jnp.float32),
        pltpu.VMEM((KT, CH), jnp.int32),
        pltpu.VMEM((CH,), jnp.float32),
        pltpu.VMEM((RPT,), jnp.float32),
        pltpu.SemaphoreType.DMA,
    ],
)
def _hist_kernel(col_hbm, out_hbm, hist_sp, col_v, ones_v, zb_v, sem):
    cid = lax.axis_index("c")
    sid = lax.axis_index("s")
    base = (cid * NS + sid) * KT

    def zb_body(i, _):
        zb_v[pl.ds(i * 16, 16)] = _zero16()
        return 0

    lax.fori_loop(0, RPT // 16, zb_body, 0)

    def ones_body(i, _):
        ones_v[pl.ds(i * 16, 16)] = jnp.ones((16,), jnp.float32)
        return 0

    lax.fori_loop(0, CH // 16, ones_body, 0)

    pltpu.sync_copy(zb_v, hist_sp.at[pl.ds(sid * RPT, RPT)])
    plsc.subcore_barrier()

    pltpu.async_copy(col_hbm.at[pl.ds(base, KT)], col_v, sem).wait()

    def chunk_body(j, _):
        pltpu.sync_copy(ones_v, hist_sp.at[col_v.at[j]], add=True)
        return 0

    lax.fori_loop(0, KT, chunk_body, 0)
    plsc.subcore_barrier()
    pltpu.sync_copy(hist_sp.at[pl.ds(sid * RPT, RPT)], out_hbm.at[cid, sid])


def _scale_body(dga_ref, dgb_ref, a_ref, b_ref, dis_ref):
    deg = dga_ref[...] + dgb_ref[...]
    dis = jnp.where(deg > 0.0, lax.rsqrt(deg), 0.0)
    dis_ref[...] = dis
    b_ref[...] = a_ref[...] * dis


_ROWS_BLK = 1000
_GRID = N // _ROWS_BLK


def _scale_call(dga, dgb, a):
    return pl.pallas_call(
        _scale_body,
        grid=(_GRID,),
        in_specs=[
            pl.BlockSpec((_ROWS_BLK, 1), lambda i: (i, 0)),
            pl.BlockSpec((_ROWS_BLK, 1), lambda i: (i, 0)),
            pl.BlockSpec((_ROWS_BLK, D), lambda i: (i, 0)),
        ],
        out_specs=[
            pl.BlockSpec((_ROWS_BLK, D), lambda i: (i, 0)),
            pl.BlockSpec((_ROWS_BLK, 1), lambda i: (i, 0)),
        ],
        out_shape=[
            jax.ShapeDtypeStruct((N, D), jnp.float32),
            jax.ShapeDtypeStruct((N, 1), jnp.float32),
        ],
    )(dga, dgb, a)


@functools.partial(
    pl.kernel,
    out_type=jax.ShapeDtypeStruct((NC, NS, RPT, D), jnp.float32),
    mesh=_mesh,
    scratch_types=[
        pltpu.VMEM_SHARED((NP, D), jnp.float32),
        pltpu.VMEM((KPART, CH), jnp.int32),
        pltpu.VMEM((KPART, CH), jnp.int32),
        pltpu.VMEM((CH, D), jnp.float32),
        pltpu.VMEM((CH, D), jnp.float32),
        pltpu.SemaphoreType.DMA,
        pltpu.SemaphoreType.DMA,
        pltpu.SemaphoreType.DMA,
    ],
)
def _scatter_kernel(row_hbm, col_hbm, b_hbm, out_hbm,
                    c_sp, row_v, col_v, g0_v, g1_v, sem0, sem1, semi):
    cid = lax.axis_index("c")
    sid = lax.axis_index("s")

    def zb_body(i, _):
        g0_v[i // 8, pl.ds((i % 8) * 16, 16)] = _zero16()
        return 0

    lax.fori_loop(0, CH * D // 16, zb_body, 0)

    def zcopy_body(t, _):
        pltpu.sync_copy(g0_v, c_sp.at[pl.ds(sid * RPT + t * CH, CH)])
        return 0

    lax.fori_loop(0, RPT // CH, zcopy_body, 0)
    plsc.subcore_barrier()

    dummy = b_hbm.at[pl.ds(0, CH)]

    def run_part(base):
        idx_copy = pltpu.async_copy(row_hbm.at[pl.ds(base, KPART)],
                                    row_v, semi)
        pltpu.async_copy(col_hbm.at[pl.ds(base, KPART)], col_v, sem1).wait()
        idx_copy.wait()

        pltpu.async_copy(b_hbm.at[row_v.at[0]], g0_v, sem0)

        def pair_body(i, _):
            j0 = 2 * i
            pltpu.async_copy(b_hbm.at[row_v.at[j0 + 1]], g1_v, sem1)
            pltpu.make_async_copy(dummy, g0_v, sem0).wait()
            pltpu.sync_copy(g0_v, c_sp.at[col_v.at[j0]], add=True)

            @pl.when(j0 + 2 < KPART)
            def _():
                pltpu.async_copy(b_hbm.at[row_v.at[j0 + 2]], g0_v, sem0)

            pltpu.make_async_copy(dummy, g1_v, sem1).wait()
            pltpu.sync_copy(g1_v, c_sp.at[col_v.at[j0 + 1]], add=True)
            return 0

        lax.fori_loop(0, KPART // 2, pair_body, 0)

    base0 = (cid * NS + sid) * KT
    for i in range(KT // KPART):
        run_part(base0 + i * KPART)

    plsc.subcore_barrier()
    pltpu.sync_copy(c_sp.at[pl.ds(sid * RPT, RPT)], out_hbm.at[cid, sid])


def _combine_body(scal_ref, a_ref, dis_ref, ca_ref, cb_ref, out_ref):
    sw = scal_ref[0]
    alpha = scal_ref[1]
    c = ca_ref[...] + cb_ref[...]
    out_ref[...] = sw * a_ref[...] + (alpha * dis_ref[...]) * c


def _combine_call(scal, a, dis, ca, cb):
    return pl.pallas_call(
        _combine_body,
        grid=(_GRID,),
        in_specs=[
            pl.BlockSpec(memory_space=pltpu.SMEM),
            pl.BlockSpec((_ROWS_BLK, D), lambda i: (i, 0)),
            pl.BlockSpec((_ROWS_BLK, 1), lambda i: (i, 0)),
            pl.BlockSpec((_ROWS_BLK, D), lambda i: (i, 0)),
            pl.BlockSpec((_ROWS_BLK, D), lambda i: (i, 0)),
        ],
        out_specs=pl.BlockSpec((_ROWS_BLK, D), lambda i: (i, 0)),
        out_shape=jax.ShapeDtypeStruct((N, D), jnp.float32),
    )(scal, a, dis, ca, cb)


@jax.jit
def kernel(edge_index, user_emb_w, item_emb_w, attention):
    a = jnp.concatenate([user_emb_w, item_emb_w], axis=0)

    w = jax.nn.softmax(attention.reshape(NUM_LAYERS + 1), axis=0)
    sw = jnp.sum(w)
    ks = jnp.arange(NUM_LAYERS + 1, dtype=jnp.float32)
    alpha = jnp.sum(w * ks)
    scal = jnp.stack([sw, alpha])

    row = edge_index[0]
    col = edge_index[1]
    pad = EPAD - E
    padrow = jnp.arange(pad, dtype=jnp.int32) % N
    padcol = N + (jnp.arange(pad, dtype=jnp.int32) % (NP - N))
    rowp = jnp.concatenate([row, padrow])
    colp = jnp.concatenate([col, padcol])
    rowf = rowp.reshape(NCHT, CH)
    colf = colp.reshape(NCHT, CH)

    deg2 = _hist_kernel(colf)
    degf = deg2.reshape(NC, NP)
    dga = degf[0].reshape(NP, 1)
    dgb = degf[1].reshape(NP, 1)

    b, dis = _scale_call(dga, dgb, a)

    c2 = _scatter_kernel(rowf, colf, b)
    cf = c2.reshape(NC, NP, D)
    ca = cf[0]
    cb = cf[1]

    final = _combine_call(scal, a, dis, ca, cb)
    return final[:NUM_USERS], final[NUM_USERS:]


# speedup vs baseline: 1.0135x; 1.0135x over previous
import functools
import jax
import jax.numpy as jnp
from jax import lax
from jax.experimental import pallas as pl
from jax.experimental.pallas import tpu as pltpu
from jax.experimental.pallas import tpu_sc as plsc

NUM_USERS = 5000
NUM_ITEMS = 5000
N = NUM_USERS + NUM_ITEMS
D = 128
NUM_LAYERS = 3
E = 320000

NC, NS = 2, 16
NW = NC * NS
CH = 128
NCHT = 2560
EPAD = NCHT * CH
KT = NCHT // NW
KPART = 40
NP = 10240
RPT = NP // NS

_mesh = plsc.VectorSubcoreMesh(
    core_axis_name="c", subcore_axis_name="s", num_cores=NC, num_subcores=NS
)


def _zero16():
    return jnp.zeros((16,), jnp.float32)


@functools.partial(
    pl.kernel,
    out_type=jax.ShapeDtypeStruct((NC, NS, RPT), jnp.float32),
    mesh=_mesh,
    scratch_types=[
        pltpu.VMEM_SHARED((NP,), jnp.float32),
        pltpu.VMEM((KT, CH), jnp.int32),
        pltpu.VMEM((CH,), jnp.float32),
        pltpu.VMEM((RPT,), jnp.float32),
        pltpu.SemaphoreType.DMA,
    ],
)
def _hist_kernel(col_hbm, out_hbm, hist_sp, col_v, ones_v, zb_v, sem):
    cid = lax.axis_index("c")
    sid = lax.axis_index("s")
    base = (cid * NS + sid) * KT

    def zb_body(i, _):
        zb_v[pl.ds(i * 16, 16)] = _zero16()
        return 0

    lax.fori_loop(0, RPT // 16, zb_body, 0)

    def ones_body(i, _):
        ones_v[pl.ds(i * 16, 16)] = jnp.ones((16,), jnp.float32)
        return 0

    lax.fori_loop(0, CH // 16, ones_body, 0)

    pltpu.sync_copy(zb_v, hist_sp.at[pl.ds(sid * RPT, RPT)])
    plsc.subcore_barrier()

    pltpu.async_copy(col_hbm.at[pl.ds(base, KT)], col_v, sem).wait()

    def chunk_body(j, _):
        pltpu.sync_copy(ones_v, hist_sp.at[col_v.at[j]], add=True)
        return 0

    lax.fori_loop(0, KT, chunk_body, 0)
    plsc.subcore_barrier()
    pltpu.sync_copy(hist_sp.at[pl.ds(sid * RPT, RPT)], out_hbm.at[cid, sid])


def _scale_body(dga_ref, dgb_ref, a_ref, b_ref, dis_ref):
    deg = dga_ref[...] + dgb_ref[...]
    dis = jnp.where(deg > 0.0, lax.rsqrt(deg), 0.0)
    dis_ref[...] = dis
    b_ref[...] = a_ref[...] * dis


_ROWS_BLK = 1000
_GRID = N // _ROWS_BLK


def _scale_call(dga, dgb, a):
    return pl.pallas_call(
        _scale_body,
        grid=(_GRID,),
        in_specs=[
            pl.BlockSpec((_ROWS_BLK, 1), lambda i: (i, 0)),
            pl.BlockSpec((_ROWS_BLK, 1), lambda i: (i, 0)),
            pl.BlockSpec((_ROWS_BLK, D), lambda i: (i, 0)),
        ],
        out_specs=[
            pl.BlockSpec((_ROWS_BLK, D), lambda i: (i, 0)),
            pl.BlockSpec((_ROWS_BLK, 1), lambda i: (i, 0)),
        ],
        out_shape=[
            jax.ShapeDtypeStruct((N, D), jnp.float32),
            jax.ShapeDtypeStruct((N, 1), jnp.float32),
        ],
    )(dga, dgb, a)


@functools.partial(
    pl.kernel,
    out_type=jax.ShapeDtypeStruct((NC, NS, RPT, D), jnp.float32),
    mesh=_mesh,
    scratch_types=[
        pltpu.VMEM_SHARED((NP, D), jnp.float32),
        pltpu.VMEM((KPART, CH), jnp.int32),
        pltpu.VMEM((KPART, CH), jnp.int32),
        pltpu.VMEM((CH, D), jnp.float32),
        pltpu.VMEM((CH, D), jnp.float32),
        pltpu.SemaphoreType.DMA,
        pltpu.SemaphoreType.DMA,
        pltpu.SemaphoreType.DMA,
    ],
)
def _scatter_kernel(row_hbm, col_hbm, b_hbm, out_hbm,
                    c_sp, row_v, col_v, g0_v, g1_v, sem0, sem1, semi):
    cid = lax.axis_index("c")
    sid = lax.axis_index("s")

    def zb_body(i, _):
        g0_v[i // 8, pl.ds((i % 8) * 16, 16)] = _zero16()
        return 0

    lax.fori_loop(0, CH * D // 16, zb_body, 0)

    def zcopy_body(t, _):
        pltpu.sync_copy(g0_v, c_sp.at[pl.ds(sid * RPT + t * CH, CH)])
        return 0

    lax.fori_loop(0, RPT // CH, zcopy_body, 0)
    plsc.subcore_barrier()

    dummy = b_hbm.at[pl.ds(0, CH)]

    def run_part(base):
        idx_copy = pltpu.async_copy(row_hbm.at[pl.ds(base, KPART)],
                                    row_v, semi)
        pltpu.async_copy(col_hbm.at[pl.ds(base, KPART)], col_v, sem1).wait()
        idx_copy.wait()

        pltpu.async_copy(b_hbm.at[row_v.at[0]], g0_v, sem0)

        def pair_body(i, _):
            j0 = 2 * i
            pltpu.async_copy(b_hbm.at[row_v.at[j0 + 1]], g1_v, sem1)
            pltpu.make_async_copy(dummy, g0_v, sem0).wait()
            pltpu.sync_copy(g0_v, c_sp.at[col_v.at[j0]], add=True)

            @pl.when(j0 + 2 < KPART)
            def _():
                pltpu.async_copy(b_hbm.at[row_v.at[j0 + 2]], g0_v, sem0)

            pltpu.make_async_copy(dummy, g1_v, sem1).wait()
            pltpu.sync_copy(g1_v, c_sp.at[col_v.at[j0 + 1]], add=True)
            return 0

        lax.fori_loop(0, KPART // 2, pair_body, 0)

    base0 = (cid * NS + sid) * KT
    for i in range(KT // KPART):
        run_part(base0 + i * KPART)

    plsc.subcore_barrier()
    pltpu.sync_copy(c_sp.at[pl.ds(sid * RPT, RPT)], out_hbm.at[cid, sid])


def _combine_body(scal_ref, a_ref, dis_ref, ca_ref, cb_ref, out_ref):
    sw = scal_ref[0]
    alpha = scal_ref[1]
    c = ca_ref[...] + cb_ref[...]
    out_ref[...] = sw * a_ref[...] + (alpha * dis_ref[...]) * c


def _combine_call(scal, a, dis, ca, cb):
    return pl.pallas_call(
        _combine_body,
        grid=(_GRID,),
        in_specs=[
            pl.BlockSpec(memory_space=pltpu.SMEM),
            pl.BlockSpec((_ROWS_BLK, D), lambda i: (i, 0)),
            pl.BlockSpec((_ROWS_BLK, 1), lambda i: (i, 0)),
            pl.BlockSpec((_ROWS_BLK, D), lambda i: (i, 0)),
            pl.BlockSpec((_ROWS_BLK, D), lambda i: (i, 0)),
        ],
        out_specs=pl.BlockSpec((_ROWS_BLK, D), lambda i: (i, 0)),
        out_shape=jax.ShapeDtypeStruct((N, D), jnp.float32),
    )(scal, a, dis, ca, cb)


@jax.jit
def kernel(edge_index, user_emb_w, item_emb_w, attention):
    a = jnp.concatenate([user_emb_w, item_emb_w], axis=0)

    w = jax.nn.softmax(attention.reshape(NUM_LAYERS + 1), axis=0)
    sw = jnp.sum(w)
    ks = jnp.arange(NUM_LAYERS + 1, dtype=jnp.float32)
    alpha = jnp.sum(w * ks)
    scal = jnp.stack([sw, alpha])

    row = edge_index[0]
    col = edge_index[1]
    pad = EPAD - E
    padrow = jnp.arange(pad, dtype=jnp.int32) % N
    padcol = N + (jnp.arange(pad, dtype=jnp.int32) % (NP - N))
    rowp = jnp.concatenate([row, padrow])
    colp = jnp.concatenate([col, padcol])
    rowf = rowp.reshape(NCHT, CH)
    colf = colp.reshape(NCHT, CH)

    deg2 = _hist_kernel(colf)
    degf = deg2.reshape(NC, NP)
    dga = degf[0, :N].reshape(N, 1)
    dgb = degf[1, :N].reshape(N, 1)

    b, dis = _scale_call(dga, dgb, a)

    c2 = _scatter_kernel(rowf, colf, b)
    cf = c2.reshape(NC, NP, D)
    ca = cf[0, :N]
    cb = cf[1, :N]

    final = _combine_call(scal, a, dis, ca, cb)
    return final[:NUM_USERS], final[NUM_USERS:]
